# baseline (speedup 1.0000x reference)
import functools

import jax
import jax.numpy as jnp
from jax import lax
from jax.experimental import pallas as pl
from jax.experimental.pallas import tpu as pltpu
from jax.experimental.pallas import tpu_sc as plsc

N_LEVEL = 26
N_EMB = 100000
D_EMB = 32
BATCH = 16384

NUM_CORES = 2
NUM_SUBCORES = 16
NW = NUM_CORES * NUM_SUBCORES
ROWS = N_LEVEL * D_EMB
RPW = ROWS // NW
LANES = 16
OCHUNK = 4096
NOC = BATCH // OCHUNK


def _emb_kernel(idx_hbm, tab_hbm, out_hbm, idx_v, row_v, out_v, sem_r, sem_o):
    wid = lax.axis_index("s") * NUM_CORES + lax.axis_index("c")
    r0 = wid * RPW
    rlast = r0 + RPW - 1

    l0 = lax.shift_right_logical(r0, 5)
    d0 = lax.bitwise_and(r0, 31)
    pltpu.async_copy(tab_hbm.at[l0, d0], row_v, sem_r)
    pltpu.sync_copy(idx_hbm.at[l0], idx_v)
    for b in range(2):
        pltpu.async_copy(
            out_hbm.at[l0, d0, pl.ds(0, OCHUNK)], out_v.at[b], sem_o)

    def row_body(j, carry):
        r = r0 + j
        l = lax.shift_right_logical(r, 5)
        d = lax.bitwise_and(r, 31)

        pltpu.make_async_copy(tab_hbm.at[l, d], row_v, sem_r).wait()

        for c in range(NOC):
            b = c % 2
            pltpu.make_async_copy(
                out_hbm.at[l, d, pl.ds(0, OCHUNK)], out_v.at[b], sem_o,
            ).wait()

            @plsc.parallel_loop(0, OCHUNK // LANES, unroll=8)
            def _(g, c=c, b=b):
                iv = idx_v[pl.ds(c * OCHUNK + g * LANES, LANES)]
                out_v[b, pl.ds(g * LANES, LANES)] = (
                    plsc.load_gather(row_v, [iv]))

            pltpu.async_copy(
                out_v.at[b],
                out_hbm.at[l, d, pl.ds(c * OCHUNK, OCHUNK)],
                sem_o,
            )

        rn = jnp.minimum(r + 1, rlast)
        ln = lax.shift_right_logical(rn, 5)
        dn = lax.bitwise_and(rn, 31)
        pltpu.async_copy(tab_hbm.at[ln, dn], row_v, sem_r)

        @pl.when(ln != l)
        def _():
            pltpu.sync_copy(idx_hbm.at[ln], idx_v)
        return carry

    lax.fori_loop(0, RPW, row_body, 0)

    llast = lax.shift_right_logical(rlast, 5)
    dlast = lax.bitwise_and(rlast, 31)
    pltpu.make_async_copy(tab_hbm.at[llast, dlast], row_v, sem_r).wait()
    for b in range(2):
        pltpu.make_async_copy(
            out_hbm.at[llast, dlast, pl.ds(0, OCHUNK)], out_v.at[b], sem_o,
        ).wait()


def kernel(idx, weight):
    tab_t = jnp.transpose(weight, (0, 2, 1))

    mesh = plsc.VectorSubcoreMesh(core_axis_name="c", subcore_axis_name="s")
    run = functools.partial(
        pl.kernel,
        mesh=mesh,
        compiler_params=pltpu.CompilerParams(needs_layout_passes=False),
        out_type=jax.ShapeDtypeStruct((N_LEVEL, D_EMB, BATCH), jnp.float32),
        scratch_types=[
            pltpu.VMEM((BATCH,), jnp.int32),
            pltpu.VMEM((N_EMB,), jnp.float32),
            pltpu.VMEM((2, OCHUNK), jnp.float32),
            pltpu.SemaphoreType.DMA,
            pltpu.SemaphoreType.DMA,
        ],
    )(_emb_kernel)
    out_t = run(idx.astype(jnp.int32), tab_t)
    return jnp.transpose(out_t, (0, 2, 1))


# speedup vs baseline: 1.0211x; 1.0211x over previous
import functools

import jax
import jax.numpy as jnp
from jax import lax
from jax.experimental import pallas as pl
from jax.experimental.pallas import tpu as pltpu
from jax.experimental.pallas import tpu_sc as plsc

N_LEVEL = 26
N_EMB = 100000
D_EMB = 32
BATCH = 16384

NUM_CORES = 2
NUM_SUBCORES = 16
NW = NUM_CORES * NUM_SUBCORES
ROWS = N_LEVEL * D_EMB
RPW = ROWS // NW
LANES = 16
HALF = BATCH // 2


def _emb_kernel(idx_hbm, tab_hbm, out_hbm, idx_v, row_v, out_v):
    wid = lax.axis_index("s") * NUM_CORES + lax.axis_index("c")

    def row_body(j, l_prev):
        r = wid * RPW + j
        l = lax.shift_right_logical(r, 5)
        d = lax.bitwise_and(r, 31)

        @pl.when(l != l_prev)
        def _():
            pltpu.sync_copy(idx_hbm.at[l], idx_v)

        pltpu.sync_copy(tab_hbm.at[l, d], row_v)

        for h in range(2):
            @plsc.parallel_loop(0, HALF // LANES, unroll=16)
            def _(g, h=h):
                iv = idx_v[pl.ds(h * HALF + g * LANES, LANES)]
                out_v[pl.ds(g * LANES, LANES)] = (
                    plsc.load_gather(row_v, [iv]))
            pltpu.sync_copy(out_v, out_hbm.at[l, d, pl.ds(h * HALF, HALF)])
        return l

    lax.fori_loop(0, RPW, row_body, jnp.int32(-1))


def kernel(idx, weight):
    tab_t = jnp.transpose(weight, (0, 2, 1))

    mesh = plsc.VectorSubcoreMesh(core_axis_name="c", subcore_axis_name="s")
    run = functools.partial(
        pl.kernel,
        mesh=mesh,
        compiler_params=pltpu.CompilerParams(needs_layout_passes=False),
        out_type=jax.ShapeDtypeStruct((N_LEVEL, D_EMB, BATCH), jnp.float32),
        scratch_types=[
            pltpu.VMEM((BATCH,), jnp.int32),
            pltpu.VMEM((N_EMB,), jnp.float32),
            pltpu.VMEM((HALF,), jnp.float32),
        ],
    )(_emb_kernel)
    out_t = run(idx.astype(jnp.int32), tab_t)
    return jnp.transpose(out_t, (0, 2, 1))
